# Initial kernel scaffold; baseline (speedup 1.0000x reference)
import jax
import jax.numpy as jnp
from jax.experimental import pallas as pl


def kernel(loc, cube):
    raise NotImplementedError("write your pallas kernel here")


# speedup vs baseline: 2.0297x; 2.0297x over previous
import functools

import jax
import jax.numpy as jnp
from jax import lax
from jax.experimental import pallas as pl
from jax.experimental.pallas import tpu as pltpu
from jax.experimental.pallas import tpu_sc as plsc

_RES = 128
_C = 16
_V = _RES * _RES * _RES
_HALF = 2.0
_N = 500000

_NC = 2
_NS = 16
_NW = _NC * _NS
_L = 16

_CH = 128
_KPW = 123
_NP = _CH * _NW * _KPW

_BT = 4096


def _prep_body(loc_ref, idx_ref, w_ref):
    g = loc_ref[...] * (1.0 / _HALF)
    ix = jnp.clip((g + 1.0) * 64.0 - 0.5, 0.0, 127.0)
    i0 = jnp.minimum(ix.astype(jnp.int32), _RES - 2)
    w1 = ix - i0.astype(jnp.float32)
    w0 = 1.0 - w1
    x0 = i0[0:1]
    y0 = i0[1:2]
    z0 = i0[2:3]
    base = z0 * (_RES * _RES) + y0 * _RES + x0
    wx = (w0[0:1], w1[0:1])
    wy = (w0[1:2], w1[1:2])
    wz = (w0[2:3], w1[2:3])
    rows_i = []
    rows_w = []
    for dz in (0, 1):
        for dy in (0, 1):
            for dx in (0, 1):
                rows_i.append(base + (dz * _RES * _RES + dy * _RES + dx))
                rows_w.append(wz[dz] * wy[dy] * wx[dx])
    idx_ref[...] = jnp.concatenate(rows_i, axis=0)
    w_ref[...] = jnp.concatenate(rows_w, axis=0)


def _prep(loc_pad):
    return pl.pallas_call(
        _prep_body,
        grid=(_NP // _BT,),
        in_specs=[pl.BlockSpec((8, _BT), lambda i: (0, i))],
        out_specs=[
            pl.BlockSpec((8, _BT), lambda i: (0, i)),
            pl.BlockSpec((8, _BT), lambda i: (0, i)),
        ],
        out_shape=[
            jax.ShapeDtypeStruct((8, _NP), jnp.int32),
            jax.ShapeDtypeStruct((8, _NP), jnp.float32),
        ],
    )(loc_pad)


_sc_mesh = plsc.VectorSubcoreMesh(core_axis_name="c", subcore_axis_name="s")


@functools.partial(
    pl.kernel,
    mesh=_sc_mesh,
    out_type=jax.ShapeDtypeStruct((_NP, _C), jnp.float32),
    scratch_types=[
        pltpu.VMEM((8, _CH), jnp.int32),
        pltpu.VMEM((8, _CH), jnp.float32),
        pltpu.VMEM((8, _CH, _L), jnp.float32),
        pltpu.VMEM((_CH, _L), jnp.float32),
        pltpu.SemaphoreType.DMA,
    ],
    compiler_params=pltpu.CompilerParams(use_tc_tiling_on_sc=False),
)
def _sc_gather(table_hbm, idx_hbm, w_hbm, out_hbm, idx_v, w_v, rows_v, out_v,
               sem):
    wid = lax.axis_index("s") * _NC + lax.axis_index("c")

    def chunk(k, carry):
        base = (wid * _KPW + k) * _CH
        pltpu.sync_copy(idx_hbm.at[:, pl.ds(base, _CH)], idx_v)
        pltpu.sync_copy(w_hbm.at[:, pl.ds(base, _CH)], w_v)
        copies = [
            pltpu.async_copy(table_hbm.at[idx_v.at[j]], rows_v.at[j], sem)
            for j in range(8)
        ]
        for cp in copies:
            cp.wait()

        def grp(g, c):
            b = g * _L
            wv = [w_v[j, pl.ds(b, _L)] for j in range(8)]
            for l in range(_L):
                i = b + l
                a0 = wv[0][l] * rows_v[0, i] + wv[1][l] * rows_v[1, i]
                a1 = wv[2][l] * rows_v[2, i] + wv[3][l] * rows_v[3, i]
                a2 = wv[4][l] * rows_v[4, i] + wv[5][l] * rows_v[5, i]
                a3 = wv[6][l] * rows_v[6, i] + wv[7][l] * rows_v[7, i]
                out_v[i] = (a0 + a1) + (a2 + a3)
            return c

        lax.fori_loop(0, _CH // _L, grp, None)
        pltpu.sync_copy(out_v, out_hbm.at[pl.ds(base, _CH)])
        return carry

    lax.fori_loop(0, _KPW, chunk, None)


def kernel(loc, cube):
    table = cube[0].reshape(_C, _V).T
    loc_pad = jnp.zeros((8, _NP), jnp.float32).at[:3, :_N].set(loc.T)
    idx, w = _prep(loc_pad)
    out = _sc_gather(table, idx, w)
    return out[:_N]


# speedup vs baseline: 3.0287x; 1.4922x over previous
import functools

import jax
import jax.numpy as jnp
from jax import lax
from jax.experimental import pallas as pl
from jax.experimental.pallas import tpu as pltpu
from jax.experimental.pallas import tpu_sc as plsc

_RES = 128
_C = 16
_V = _RES * _RES * _RES
_N = 500000

_NC = 2
_NS = 16
_NW = _NC * _NS
_L = 16

_CH = 128
_NCHUNK = -(-_N // _CH)
_KB = -(-_NCHUNK // _NW)
_TAIL_BASE = (_NCHUNK - 1) * _CH
_TAIL = _N - _TAIL_BASE
_NL = _NCHUNK * _CH

_TY = 16
_TVOX = _TY * _RES
_TPERZ = _RES // _TY
_TK = (_RES * _TPERZ) // _NW

_mesh = plsc.VectorSubcoreMesh(core_axis_name="c", subcore_axis_name="s")
_sc_params = pltpu.CompilerParams(use_tc_tiling_on_sc=False,
                                  needs_layout_passes=False)


@functools.partial(
    pl.kernel,
    mesh=_mesh,
    out_type=jax.ShapeDtypeStruct((_V, _C), jnp.float32),
    scratch_types=[
        pltpu.VMEM((_C, _TY, _RES), jnp.float32),
        pltpu.VMEM((_TVOX, _C), jnp.float32),
        pltpu.SemaphoreType.DMA,
    ],
    compiler_params=_sc_params,
)
def _sc_transpose(cube_hbm, table_hbm, in_v, out_v, sem):
    wid = lax.axis_index("s") * _NC + lax.axis_index("c")
    lanes = jnp.arange(_L, dtype=jnp.int32)

    def chunk(k, carry):
        c = wid * _TK + k
        z = c // _TPERZ
        y0 = (c % _TPERZ) * _TY
        copies = [
            pltpu.async_copy(cube_hbm.at[0, f, z, pl.ds(y0, _TY)],
                             in_v.at[f], sem)
            for f in range(_C)
        ]
        for cp in copies:
            cp.wait()

        def yrow(yy, cy):
            def xgrp(xg, cx):
                rows = yy * _RES + xg * _L + lanes
                for f in range(_C):
                    vals = in_v[f, yy, pl.ds(xg * _L, _L)]
                    plsc.store_scatter(
                        out_v, [rows, jnp.full((_L,), f, jnp.int32)], vals)
                return cx

            lax.fori_loop(0, _RES // _L, xgrp, None)
            return cy

        lax.fori_loop(0, _TY, yrow, None)
        pltpu.sync_copy(out_v, table_hbm.at[pl.ds(c * _TVOX, _TVOX)])
        return carry

    lax.fori_loop(0, _TK, chunk, None)


_OFFS = [dz * _RES * _RES + dy * _RES + dx
         for dz in (0, 1) for dy in (0, 1) for dx in (0, 1)]


@functools.partial(
    pl.kernel,
    mesh=_mesh,
    out_type=jax.ShapeDtypeStruct((_N, _C), jnp.float32),
    scratch_types=[
        pltpu.VMEM((_CH,), jnp.float32),
        pltpu.VMEM((_CH,), jnp.float32),
        pltpu.VMEM((_CH,), jnp.float32),
        pltpu.VMEM((8, _CH), jnp.int32),
        pltpu.VMEM((8, _CH), jnp.float32),
        pltpu.VMEM((8, _CH, _L), jnp.float32),
        pltpu.VMEM((_CH, _L), jnp.float32),
        pltpu.SemaphoreType.DMA,
    ],
    compiler_params=_sc_params,
)
def _sc_sample(table_hbm, lx_hbm, ly_hbm, lz_hbm, out_hbm,
               lx_v, ly_v, lz_v, idx_v, w_v, rows_v, out_v, sem):
    wid = lax.axis_index("s") * _NC + lax.axis_index("c")

    def axis_prep(lv, g):
        i = jnp.clip(lv[pl.ds(g * _L, _L)] * 32.0 + 63.5, 0.0, 127.0)
        i0 = jnp.minimum(i.astype(jnp.int32), _RES - 2)
        f1 = i - i0.astype(jnp.float32)
        return i0, 1.0 - f1, f1

    def chunk(k, carry):
        c = k * _NW + wid

        @pl.when(c < _NCHUNK)
        def _():
            base = c * _CH
            pltpu.sync_copy(lx_hbm.at[pl.ds(base, _CH)], lx_v)
            pltpu.sync_copy(ly_hbm.at[pl.ds(base, _CH)], ly_v)
            pltpu.sync_copy(lz_hbm.at[pl.ds(base, _CH)], lz_v)

            def prep(g, cp):
                x0, wx0, wx1 = axis_prep(lx_v, g)
                y0, wy0, wy1 = axis_prep(ly_v, g)
                z0, wz0, wz1 = axis_prep(lz_v, g)
                vbase = (z0 * _RES + y0) * _RES + x0
                wz_ = (wz0, wz1)
                wy_ = (wy0, wy1)
                wx_ = (wx0, wx1)
                for j, off in enumerate(_OFFS):
                    idx_v[j, pl.ds(g * _L, _L)] = vbase + off
                    dz, dy, dx = j >> 2, (j >> 1) & 1, j & 1
                    w_v[j, pl.ds(g * _L, _L)] = wz_[dz] * wy_[dy] * wx_[dx]
                return cp

            lax.fori_loop(0, _CH // _L, prep, None)

            copies = [
                pltpu.async_copy(table_hbm.at[idx_v.at[j]], rows_v.at[j], sem)
                for j in range(8)
            ]
            for cp in copies:
                cp.wait()

            def grp(g, cg):
                b = g * _L
                wv = [w_v[j, pl.ds(b, _L)] for j in range(8)]
                for l in range(_L):
                    i = b + l
                    a0 = wv[0][l] * rows_v[0, i] + wv[1][l] * rows_v[1, i]
                    a1 = wv[2][l] * rows_v[2, i] + wv[3][l] * rows_v[3, i]
                    a2 = wv[4][l] * rows_v[4, i] + wv[5][l] * rows_v[5, i]
                    a3 = wv[6][l] * rows_v[6, i] + wv[7][l] * rows_v[7, i]
                    out_v[i] = (a0 + a1) + (a2 + a3)
                return cg

            lax.fori_loop(0, _CH // _L, grp, None)

            @pl.when(c < _NCHUNK - 1)
            def _():
                pltpu.sync_copy(out_v, out_hbm.at[pl.ds(base, _CH)])

            @pl.when(c == _NCHUNK - 1)
            def _():
                pltpu.sync_copy(out_v.at[pl.ds(0, _TAIL)],
                                out_hbm.at[pl.ds(_TAIL_BASE, _TAIL)])

        return carry

    lax.fori_loop(0, _KB, chunk, None)


def kernel(loc, cube):
    zpad = jnp.zeros((_NL - _N,), jnp.float32)
    lx = jnp.concatenate([loc[:, 0], zpad])
    ly = jnp.concatenate([loc[:, 1], zpad])
    lz = jnp.concatenate([loc[:, 2], zpad])
    table = _sc_transpose(cube)
    return _sc_sample(table, lx, ly, lz)


# speedup vs baseline: 5.0635x; 1.6718x over previous
import functools

import jax
import jax.numpy as jnp
from jax import lax
from jax.experimental import pallas as pl
from jax.experimental.pallas import tpu as pltpu
from jax.experimental.pallas import tpu_sc as plsc

_RES = 128
_C = 16
_V = _RES * _RES * _RES
_N = 500000

_NC = 2
_NS = 16
_NW = _NC * _NS
_L = 16

_CH = 128
_NCHUNK = -(-_N // _CH)
_KB = -(-_NCHUNK // _NW)
_TAIL_ROW = (_NCHUNK - 1) * _CH * _C // _RES
_TAIL_R = (_N * _C // _RES) - _TAIL_ROW
_NL = _NCHUNK * _CH
_OROWS = _N * _C // _RES

_TY = 8
_TVOX = _TY * _RES
_TPERZ = _RES // _TY
_TK = (_RES * _TPERZ) // _NW

_mesh = plsc.VectorSubcoreMesh(core_axis_name="c", subcore_axis_name="s")
_sc_params = pltpu.CompilerParams(use_tc_tiling_on_sc=False,
                                  needs_layout_passes=False)


@functools.partial(
    pl.kernel,
    mesh=_mesh,
    out_type=jax.ShapeDtypeStruct((_V, _C), jnp.float32),
    scratch_types=[
        pltpu.VMEM((2, _C, _TY, _RES), jnp.float32),
        pltpu.VMEM((2, _TVOX, _C), jnp.float32),
        pltpu.SemaphoreType.DMA,
        pltpu.SemaphoreType.DMA,
        pltpu.SemaphoreType.DMA,
        pltpu.SemaphoreType.DMA,
    ],
    compiler_params=_sc_params,
)
def _sc_transpose(cube_hbm, table_hbm, in_v, out_v, si0, si1, so0, so1):
    wid = lax.axis_index("s") * _NC + lax.axis_index("c")
    lanes = jnp.arange(_L, dtype=jnp.int32)
    si = (si0, si1)
    so = (so0, so1)

    def in_descs(c, b):
        z = c // _TPERZ
        y0 = (c % _TPERZ) * _TY
        return [
            pltpu.make_async_copy(cube_hbm.at[0, f, z, pl.ds(y0, _TY)],
                                  in_v.at[b, f], si[b])
            for f in range(_C)
        ]

    def out_desc(c, b):
        return pltpu.make_async_copy(
            out_v.at[b], table_hbm.at[pl.ds(c * _TVOX, _TVOX)], so[b])

    def compute(b):
        def yrow(yy, cy):
            def xgrp(xg, cx):
                rows = yy * _RES + xg * _L + lanes
                for f in range(_C):
                    vals = in_v[b, f, yy, pl.ds(xg * _L, _L)]
                    plsc.store_scatter(
                        out_v.at[b],
                        [rows, jnp.full((_L,), f, jnp.int32)], vals)
                return cx

            lax.fori_loop(0, _RES // _L, xgrp, None)
            return cy

        lax.fori_loop(0, _TY, yrow, None)

    for b in (0, 1):
        for cp in in_descs(wid * _TK + b, b):
            cp.start()

    def pair(k2, carry):
        for b in (0, 1):
            k = k2 * 2 + b
            c = wid * _TK + k
            for cp in in_descs(c, b):
                cp.wait()

            @pl.when(k2 > 0)
            def _():
                out_desc(c - 2, b).wait()

            compute(b)
            out_desc(c, b).start()

            @pl.when(k + 2 < _TK)
            def _():
                for cp in in_descs(c + 2, b):
                    cp.start()

        return carry

    lax.fori_loop(0, _TK // 2, pair, None)
    out_desc(wid * _TK + _TK - 2, 0).wait()
    out_desc(wid * _TK + _TK - 1, 1).wait()


_OFFS = [dz * _RES * _RES + dy * _RES + dx
         for dz in (0, 1) for dy in (0, 1) for dx in (0, 1)]


@functools.partial(
    pl.kernel,
    mesh=_mesh,
    out_type=jax.ShapeDtypeStruct((_OROWS, _RES), jnp.float32),
    scratch_types=[
        pltpu.VMEM((2, 3, _CH), jnp.float32),
        pltpu.VMEM((2, 8, _CH), jnp.int32),
        pltpu.VMEM((2, 8, _CH), jnp.float32),
        pltpu.VMEM((2, 8, _CH, _L), jnp.float32),
        pltpu.VMEM((2, _CH * _C // _RES, _RES), jnp.float32),
        pltpu.SemaphoreType.DMA,
        pltpu.SemaphoreType.DMA,
        pltpu.SemaphoreType.DMA,
        pltpu.SemaphoreType.DMA,
        pltpu.SemaphoreType.DMA,
        pltpu.SemaphoreType.DMA,
    ],
    compiler_params=_sc_params,
)
def _sc_sample(table_hbm, lx_hbm, ly_hbm, lz_hbm, out_hbm,
               loc_v, idx_v, w_v, rows_v, out_v,
               sl0, sl1, sg0, sg1, so0, so1):
    wid = lax.axis_index("s") * _NC + lax.axis_index("c")
    sl = (sl0, sl1)
    sg = (sg0, sg1)
    so = (so0, so1)
    locs = (lx_hbm, ly_hbm, lz_hbm)

    def loc_descs(c, b):
        return [
            pltpu.make_async_copy(locs[a].at[pl.ds(c * _CH, _CH)],
                                  loc_v.at[b, a], sl[b])
            for a in range(3)
        ]

    def gather_descs(b):
        return [
            pltpu.make_async_copy(table_hbm.at[idx_v.at[b, j]],
                                  rows_v.at[b, j], sg[b])
            for j in range(8)
        ]

    def out_start_wait(c, b, start):
        @pl.when(c < _NCHUNK - 1)
        def _():
            d = pltpu.make_async_copy(
                out_v.at[b],
                out_hbm.at[pl.ds(c * (_CH * _C // _RES), _CH * _C // _RES)],
                so[b])
            d.start() if start else d.wait()

        @pl.when(c == _NCHUNK - 1)
        def _():
            d = pltpu.make_async_copy(
                out_v.at[b, pl.ds(0, _TAIL_R)],
                out_hbm.at[pl.ds(_TAIL_ROW, _TAIL_R)], so[b])
            d.start() if start else d.wait()

    def axis_prep(b, a, g):
        i = jnp.clip(loc_v[b, a, pl.ds(g * _L, _L)] * 32.0 + 63.5,
                     0.0, 127.0)
        i0 = jnp.minimum(i.astype(jnp.int32), _RES - 2)
        f1 = i - i0.astype(jnp.float32)
        return i0, 1.0 - f1, f1

    def prep(b):
        def grp(g, cp):
            x0, wx0, wx1 = axis_prep(b, 0, g)
            y0, wy0, wy1 = axis_prep(b, 1, g)
            z0, wz0, wz1 = axis_prep(b, 2, g)
            vbase = (z0 * _RES + y0) * _RES + x0
            wz_ = (wz0, wz1)
            wy_ = (wy0, wy1)
            wx_ = (wx0, wx1)
            for j, off in enumerate(_OFFS):
                idx_v[b, j, pl.ds(g * _L, _L)] = vbase + off
                dz, dy, dx = j >> 2, (j >> 1) & 1, j & 1
                w_v[b, j, pl.ds(g * _L, _L)] = wz_[dz] * wy_[dy] * wx_[dx]
            return cp

        lax.fori_loop(0, _CH // _L, grp, None)

    def accum(b):
        def grp(g, cg):
            bb = g * _L
            wv = [w_v[b, j, pl.ds(bb, _L)] for j in range(8)]
            for l in range(_L):
                i = bb + l
                a0 = wv[0][l] * rows_v[b, 0, i] + wv[1][l] * rows_v[b, 1, i]
                a1 = wv[2][l] * rows_v[b, 2, i] + wv[3][l] * rows_v[b, 3, i]
                a2 = wv[4][l] * rows_v[b, 4, i] + wv[5][l] * rows_v[b, 5, i]
                a3 = wv[6][l] * rows_v[b, 6, i] + wv[7][l] * rows_v[b, 7, i]
                out_v[b, 2 * g + l // 8, pl.ds((l % 8) * _L, _L)] = \
                    (a0 + a1) + (a2 + a3)
            return cg

        lax.fori_loop(0, _CH // _L, grp, None)

    c0 = wid
    for cp in loc_descs(c0, 0):
        cp.start()
    for cp in loc_descs(c0, 0):
        cp.wait()
    prep(0)
    for cp in gather_descs(0):
        cp.start()
    for cp in loc_descs(_NW + wid, 1):
        cp.start()

    def pairbody(k2, carry):
        for b in (0, 1):
            k = k2 * 2 + b
            nb = 1 - b
            c = k * _NW + wid
            cn = c + _NW
            cnn = c + 2 * _NW

            @pl.when(cn < _NCHUNK)
            def _():
                for cp in loc_descs(cn, nb):
                    cp.wait()
                prep(nb)
                for cp in gather_descs(nb):
                    cp.start()

            @pl.when(cnn < _NCHUNK)
            def _():
                for cp in loc_descs(cnn, b):
                    cp.start()

            @pl.when(c < _NCHUNK)
            def _():
                for cp in gather_descs(b):
                    cp.wait()

                @pl.when(k >= 2)
                def _():
                    out_start_wait(c - 2 * _NW, b, False)

                accum(b)
                out_start_wait(c, b, True)

        return carry

    lax.fori_loop(0, (_KB + 1) // 2, pairbody, None)

    for kk in (_KB - 3, _KB - 2, _KB - 1):
        ct = kk * _NW + wid

        @pl.when((ct < _NCHUNK) & (ct + 2 * _NW >= _NCHUNK))
        def _():
            out_start_wait(ct, kk % 2, False)


def kernel(loc, cube):
    zpad = jnp.zeros((_NL - _N,), jnp.float32)
    lx = jnp.concatenate([loc[:, 0], zpad])
    ly = jnp.concatenate([loc[:, 1], zpad])
    lz = jnp.concatenate([loc[:, 2], zpad])
    table = _sc_transpose(cube)
    out = _sc_sample(table, lx, ly, lz)
    return out.reshape(_N, _C)
